# baseline (speedup 1.0000x reference)
import dataclasses
import functools

import jax
import jax.numpy as jnp
from jax import lax
from jax.experimental import pallas as pl
from jax.experimental.pallas import tpu as pltpu
from jax.experimental.pallas import tpu_sc as plsc

N = 10000
E = 320000
F_IN = 128
HID = 64
C_PAD = 16
R = 4

NS = 10240
RN = R * NS
DUMMY_KEY = RN
K_SEG = 40064
DUMMY_SEG = R * N
NTILES = 32
CHUNK = 128
NCHUNK = 80
NCH0 = 80
NCH1 = 80
E_PAD = NTILES * NCHUNK * CHUNK
EROWS = E_PAD // CHUNK

_HIGH = lax.Precision.HIGHEST
_mesh = plsc.VectorSubcoreMesh(core_axis_name="c", subcore_axis_name="s",
                               num_cores=2, num_subcores=16)
_sc_params = pltpu.CompilerParams()
if "needs_layout_passes" in pltpu.CompilerParams.__dataclass_fields__:
    _sc_params = dataclasses.replace(_sc_params, needs_layout_passes=False)
if "use_tc_tiling_on_sc" in pltpu.CompilerParams.__dataclass_fields__:
    _sc_params = dataclasses.replace(_sc_params, use_tc_tiling_on_sc=False)


def _wprep_body(w_ref, at_ref, wsc_ref):
    cols = []
    for r in range(R):
        cols.append(jnp.dot(w_ref[r], at_ref[r],
                            preferred_element_type=jnp.float32,
                            precision=_HIGH))
    wsc_ref[...] = jnp.concatenate(cols, axis=1)


def _wprep(w, at, fin):
    return pl.pallas_call(
        _wprep_body,
        out_shape=jax.ShapeDtypeStruct((fin, 2 * R), jnp.float32),
    )(w, at)


def _prep_body(x_ref, w_ref, wsc_ref, root_ref, bias_ref,
               xw_ref, sij_ref, xroot_ref):
    xb = x_ref[...]
    for r in range(R):
        xw_ref[r] = jnp.dot(xb, w_ref[r], preferred_element_type=jnp.float32,
                            precision=_HIGH)
    sij_ref[...] = jnp.dot(xb, wsc_ref[...],
                           preferred_element_type=jnp.float32,
                           precision=_HIGH)
    xroot_ref[...] = (jnp.dot(xb, root_ref[...],
                              preferred_element_type=jnp.float32,
                              precision=_HIGH) + bias_ref[...])


def _prep(x, w, wsc, root, bias2, fin, fo):
    nb = NS // 1024
    return pl.pallas_call(
        _prep_body,
        grid=(nb,),
        in_specs=[
            pl.BlockSpec((1024, fin), lambda i: (i, 0)),
            pl.BlockSpec((R, fin, fo), lambda i: (0, 0, 0)),
            pl.BlockSpec((fin, 2 * R), lambda i: (0, 0)),
            pl.BlockSpec((fin, fo), lambda i: (0, 0)),
            pl.BlockSpec((1, fo), lambda i: (0, 0)),
        ],
        out_specs=[
            pl.BlockSpec((R, 1024, fo), lambda i: (0, i, 0)),
            pl.BlockSpec((1024, 2 * R), lambda i: (i, 0)),
            pl.BlockSpec((1024, fo), lambda i: (i, 0)),
        ],
        out_shape=[
            jax.ShapeDtypeStruct((R, NS, fo), jnp.float32),
            jax.ShapeDtypeStruct((NS, 2 * R), jnp.float32),
            jax.ShapeDtypeStruct((NS, fo), jnp.float32),
        ],
    )(x, w, wsc, root, bias2)


def _mid1_body(sa_ref, sb_ref, inv_ref, cnt_ref):
    a = sa_ref[...] + sb_ref[...]
    nbr = pltpu.roll(a, 127, 1)
    cntc = jnp.maximum(nbr, 1.0)
    inv_ref[...] = 1.0 / (a * cntc + 1e-30)
    cnt_ref[...] = cntc


def _mid1(sa, sb):
    shp = jax.ShapeDtypeStruct((2 * K_SEG // 128, 128), jnp.float32)
    return pl.pallas_call(_mid1_body, out_shape=[shp, shp])(sa, sb)


def _mid2_body(sa_ref, sb_ref, cnt_ref, inv_ref):
    a = sa_ref[...] + sb_ref[...]
    inv_ref[...] = 1.0 / (a * cnt_ref[...] + 1e-30)


def _mid2(sa, sb, cntc):
    shp = jax.ShapeDtypeStruct((2 * K_SEG // 128, 128), jnp.float32)
    return pl.pallas_call(_mid2_body, out_shape=shp)(sa, sb, cntc)


def _post1prep2_body(aa_ref, ab_ref, xr_ref, w_ref, wsc_ref, root_ref,
                     bias_ref, xw_ref, sij_ref, xroot_ref):
    h = jnp.maximum(aa_ref[...] + ab_ref[...] + xr_ref[...], 0.0)
    for r in range(R):
        xw_ref[r] = jnp.dot(h, w_ref[r], preferred_element_type=jnp.float32,
                            precision=_HIGH)
    sij_ref[...] = jnp.dot(h, wsc_ref[...],
                           preferred_element_type=jnp.float32,
                           precision=_HIGH)
    xroot_ref[...] = (jnp.dot(h, root_ref[...],
                              preferred_element_type=jnp.float32,
                              precision=_HIGH) + bias_ref[...])


def _post1prep2(agg, xroot, w, wsc, root, bias2):
    nb = NS // 1024
    fo = C_PAD
    spec_h = pl.BlockSpec((1024, HID), lambda i: (i, 0))
    return pl.pallas_call(
        _post1prep2_body,
        grid=(nb,),
        in_specs=[
            spec_h, spec_h, spec_h,
            pl.BlockSpec((R, HID, fo), lambda i: (0, 0, 0)),
            pl.BlockSpec((HID, 2 * R), lambda i: (0, 0)),
            pl.BlockSpec((HID, fo), lambda i: (0, 0)),
            pl.BlockSpec((1, fo), lambda i: (0, 0)),
        ],
        out_specs=[
            pl.BlockSpec((R, 1024, fo), lambda i: (0, i, 0)),
            pl.BlockSpec((1024, 2 * R), lambda i: (i, 0)),
            pl.BlockSpec((1024, fo), lambda i: (i, 0)),
        ],
        out_shape=[
            jax.ShapeDtypeStruct((R, NS, fo), jnp.float32),
            jax.ShapeDtypeStruct((NS, 2 * R), jnp.float32),
            jax.ShapeDtypeStruct((NS, fo), jnp.float32),
        ],
    )(agg[0], agg[1], xroot, w, wsc, root, bias2)


def _post2_body(a_ref, b_ref, xr_ref, o_ref):
    o = a_ref[...] + b_ref[...] + xr_ref[...]
    col = lax.broadcasted_iota(jnp.int32, o.shape, 1)
    om = jnp.where(col < 10, o, -1e30)
    m = jnp.max(om, axis=1, keepdims=True)
    s = jnp.sum(jnp.exp(om - m), axis=1, keepdims=True)
    o_ref[...] = o - m - jnp.log(s)


def _post2(agg, xroot):
    nb = NS // 1024
    spec = pl.BlockSpec((1024, C_PAD), lambda i: (i, 0))
    return pl.pallas_call(
        _post2_body,
        grid=(nb,),
        in_specs=[spec, spec, spec],
        out_specs=spec,
        out_shape=jax.ShapeDtypeStruct((NS, C_PAD), jnp.float32),
    )(agg[0], agg[1], xroot)


_ZERO16 = functools.partial(jnp.zeros, (16,), jnp.float32)
TILE_E = NCHUNK * CHUNK


def _sc_pass1_body(tbl_hbm, ki_hbm, kj_hbm, ks_hbm, z_hbm, stats_out, ex_out,
                   kib, kjb, ksb, ssb, sib, sjb, exb, rows, sems):
    cid = lax.axis_index("c")
    sid = lax.axis_index("s")
    wid = cid * 16 + sid
    semi, semg, semsc = sems

    pltpu.sync_copy(z_hbm, rows.at[0])

    @pl.loop(0, 19)
    def _(i):
        pltpu.sync_copy(rows.at[0], stats_out.at[pl.ds(sid * 2504 + i * CHUNK,
                                                       CHUNK)])
    pltpu.sync_copy(rows.at[0].at[pl.ds(0, 72)],
                    stats_out.at[pl.ds(sid * 2504 + 19 * CHUNK, 72)])
    plsc.subcore_barrier()

    ones16 = jnp.ones((16,), jnp.float32)
    col0 = jnp.zeros((16,), jnp.int32)
    col1 = jnp.ones((16,), jnp.int32)
    base_rows = lax.iota(jnp.int32, 16)

    def pipeline(nch, tbase):
        def idx_copies(jj, b):
            bs = pl.ds(tbase + jj * CHUNK, CHUNK)
            return [(ki_hbm.at[bs], kib.at[b], semi[b]),
                    (kj_hbm.at[bs], kjb.at[b], semi[b]),
                    (ks_hbm.at[bs], ksb.at[b], semi[b])]

        def g_copies(b):
            return [(tbl_hbm.at[kib.at[b]], sib.at[b], semg[b]),
                    (tbl_hbm.at[kjb.at[b]], sjb.at[b], semg[b])]

        def fire(copies, **kw):
            for src, dst, sm in copies:
                pltpu.async_copy(src, dst, sm, **kw)

        def drain(copies, **kw):
            for src, dst, sm in copies:
                pltpu.make_async_copy(src, dst, sm).wait()

        def s_copy(b):
            return [(rows.at[b], stats_out.at[ssb.at[b]], semsc[b])]

        def compute(j, b):
            for q in range(CHUNK // 16):
                sl = pl.ds(q * 16, 16)
                a = sib[b, sl] + sjb[b, sl]
                a = jnp.where(a >= 0.0, a, 0.2 * a)
                a = jnp.clip(a, -60.0, 60.0)
                e = jnp.exp(a)
                exb[pl.ds(j * CHUNK + q * 16, 16)] = e
                ssb[b, sl] = ksb[b, sl]
                ridx = base_rows + q * 16
                plsc.store_scatter(rows.at[b], [ridx, col0], e)
                plsc.store_scatter(rows.at[b], [ridx, col1], ones16)

        fire(idx_copies(0, 0))
        drain(idx_copies(0, 0))
        fire(g_copies(0))
        fire(idx_copies(1, 1))

        @pl.loop(0, nch // 2)
        def _(pp):
            for b in (0, 1):
                j = pp * 2 + b
                drain(g_copies(b))
                if b == 0:
                    @pl.when(pp >= 1)
                    def _():
                        drain(s_copy(1))
                    drain(idx_copies(j + 1, 1))
                    fire(g_copies(1))
                else:
                    @pl.when(pp < nch // 2 - 1)
                    def _():
                        drain(s_copy(0))
                        drain(idx_copies(j + 1, 0))
                        fire(g_copies(0))
                compute(j, b)
                fire(s_copy(b), add=True)

                @pl.when(pp < nch // 2 - 1)
                def _():
                    fire(idx_copies(j + 2, b))

        drain(s_copy(0))
        drain(s_copy(1))
        pltpu.sync_copy(exb.at[pl.ds(0, nch * CHUNK)],
                        ex_out.at[pl.ds(tbase, nch * CHUNK)])

    @pl.when(cid == 0)
    def _():
        pipeline(NCH0, (sid * NCH0) * CHUNK)

    @pl.when(cid == 1)
    def _():
        pipeline(NCH1, (16 * NCH0 + sid * NCH1) * CHUNK)

    plsc.subcore_barrier()


def _sc_pass1(tbl, ki, kj, ks, zrows):
    @functools.partial(
        pl.kernel,
        out_type=[jax.ShapeDtypeStruct((2, K_SEG, 2), jnp.float32),
                  jax.ShapeDtypeStruct((E_PAD,), jnp.float32)],
        mesh=_mesh,
        scratch_types=[
            pltpu.VMEM((2, CHUNK), jnp.int32),
            pltpu.VMEM((2, CHUNK), jnp.int32),
            pltpu.VMEM((2, CHUNK), jnp.int32),
            pltpu.VMEM((2, CHUNK), jnp.int32),
            pltpu.VMEM((2, CHUNK), jnp.float32),
            pltpu.VMEM((2, CHUNK), jnp.float32),
            pltpu.VMEM((NCH0 * CHUNK,), jnp.float32),
            pltpu.VMEM((2, CHUNK, 2), jnp.float32),
            pltpu.SemaphoreType.DMA,
            pltpu.SemaphoreType.DMA,
            pltpu.SemaphoreType.DMA,
            pltpu.SemaphoreType.DMA,
            pltpu.SemaphoreType.DMA,
            pltpu.SemaphoreType.DMA,
            pltpu.VMEM_SHARED((K_SEG, 2), jnp.float32),
        ],
        compiler_params=_sc_params,
    )
    def run(tbl_hbm, ki_hbm, kj_hbm, ks_hbm, z_hbm, stats_hbm, ex_hbm,
            kib, kjb, ksb, ssb, sib, sjb, exb, rows,
            si0, si1, sg0, sg1, ss0, ss1, stats_sh):
        cid = lax.axis_index("c")
        sid = lax.axis_index("s")
        sems = ([si0, si1], [sg0, sg1], [ss0, ss1])
        _sc_pass1_body(tbl_hbm, ki_hbm, kj_hbm, ks_hbm, z_hbm, stats_sh,
                       ex_hbm, kib, kjb, ksb, ssb, sib, sjb, exb, rows, sems)

        @pl.when(sid == 0)
        def _():
            pltpu.sync_copy(stats_sh, stats_hbm.at[cid])

    return run(tbl, ki, kj, ks, zrows)


def _sc_pass2_body(inv_hbm, ks_hbm, kj_hbm, d_hbm, ew_hbm, ex_hbm, xw_hbm,
                   agg_out, ksb, kjb, db, sdb, ewc, exc, invc, coefb, rows,
                   sems, fo):
    cid = lax.axis_index("c")
    sid = lax.axis_index("s")
    wid = cid * 16 + sid
    nt = fo // 16
    semi, semg, semsc = sems

    @pl.loop(0, CHUNK)
    def _(i):
        for t in range(nt):
            rows[0, i, pl.ds(t * 16, 16)] = _ZERO16()

    @pl.loop(0, NS // 16 // CHUNK)
    def _(i):
        pltpu.sync_copy(rows.at[0],
                        agg_out.at[pl.ds(sid * (NS // 16) + i * CHUNK, CHUNK)])
    plsc.subcore_barrier()

    def pipeline(nch, tbase):
        def idx_copies(jj, b):
            bs = pl.ds(tbase + jj * CHUNK, CHUNK)
            return [(ks_hbm.at[bs], ksb.at[b], semi[b]),
                    (kj_hbm.at[bs], kjb.at[b], semi[b]),
                    (d_hbm.at[bs], db.at[b], semi[b]),
                    (ew_hbm.at[bs], ewc.at[b], semi[b]),
                    (ex_hbm.at[bs], exc.at[b], semi[b])]

        def g_copies(b):
            return [(inv_hbm.at[ksb.at[b]], invc.at[b], semg[b]),
                    (xw_hbm.at[kjb.at[b]], rows.at[b], semg[b])]

        def fire(copies, **kw):
            for src, dst, sm in copies:
                pltpu.async_copy(src, dst, sm, **kw)

        def drain(copies, **kw):
            for src, dst, sm in copies:
                pltpu.make_async_copy(src, dst, sm).wait()

        def s_copy(b):
            return [(rows.at[b], agg_out.at[sdb.at[b]], semsc[b])]

        def compute(b):
            for q in range(CHUNK // 16):
                sl = pl.ds(q * 16, 16)
                coefb[sl] = ewc[b, sl] * exc[b, sl] * invc[b, sl]
                sdb[b, sl] = db[b, sl]
            for jj in range(CHUNK):
                cv = plsc.load_gather(coefb, [jnp.full((16,), jj, jnp.int32)])
                for t in range(nt):
                    slt = pl.ds(t * 16, 16)
                    rows[b, jj, slt] = rows[b, jj, slt] * cv

        fire(idx_copies(0, 0))
        drain(idx_copies(0, 0))
        fire(g_copies(0))
        fire(idx_copies(1, 1))

        @pl.loop(0, nch // 2)
        def _(pp):
            for b in (0, 1):
                j = pp * 2 + b
                drain(g_copies(b))
                if b == 0:
                    @pl.when(pp >= 1)
                    def _():
                        drain(s_copy(1))
                    drain(idx_copies(j + 1, 1))
                    fire(g_copies(1))
                else:
                    @pl.when(pp < nch // 2 - 1)
                    def _():
                        drain(s_copy(0))
                        drain(idx_copies(j + 1, 0))
                        fire(g_copies(0))
                compute(b)
                fire(s_copy(b), add=True)

                @pl.when(pp < nch // 2 - 1)
                def _():
                    fire(idx_copies(j + 2, b))

        drain(s_copy(0))
        drain(s_copy(1))

    @pl.when(cid == 0)
    def _():
        pipeline(NCH0, (sid * NCH0) * CHUNK)

    @pl.when(cid == 1)
    def _():
        pipeline(NCH1, (16 * NCH0 + sid * NCH1) * CHUNK)

    plsc.subcore_barrier()


def _sc_pass2(inv, ks, kj, d, ew, ex, xw, fo):
    @functools.partial(
        pl.kernel,
        out_type=jax.ShapeDtypeStruct((2, NS, fo), jnp.float32),
        mesh=_mesh,
        scratch_types=[
            pltpu.VMEM((2, CHUNK), jnp.int32),
            pltpu.VMEM((2, CHUNK), jnp.int32),
            pltpu.VMEM((2, CHUNK), jnp.int32),
            pltpu.VMEM((2, CHUNK), jnp.int32),
            pltpu.VMEM((2, CHUNK), jnp.float32),
            pltpu.VMEM((2, CHUNK), jnp.float32),
            pltpu.VMEM((2, CHUNK), jnp.float32),
            pltpu.VMEM((CHUNK,), jnp.float32),
            pltpu.VMEM((2, CHUNK, fo), jnp.float32),
            pltpu.SemaphoreType.DMA,
            pltpu.SemaphoreType.DMA,
            pltpu.SemaphoreType.DMA,
            pltpu.SemaphoreType.DMA,
            pltpu.SemaphoreType.DMA,
            pltpu.SemaphoreType.DMA,
            pltpu.VMEM_SHARED((NS, fo), jnp.float32),
        ],
        compiler_params=_sc_params,
    )
    def run(inv_hbm, ks_hbm, kj_hbm, d_hbm, ew_hbm, ex_hbm, xw_hbm, agg_hbm,
            ksb, kjb, db, sdb, ewc, exc, invc, coefb, rows,
            si0, si1, sg0, sg1, ss0, ss1, agg_sh):
        cid = lax.axis_index("c")
        sid = lax.axis_index("s")
        sems = ([si0, si1], [sg0, sg1], [ss0, ss1])
        _sc_pass2_body(inv_hbm, ks_hbm, kj_hbm, d_hbm, ew_hbm, ex_hbm,
                       xw_hbm, agg_sh, ksb, kjb, db, sdb, ewc, exc, invc,
                       coefb, rows, sems, fo)

        @pl.when(sid == 0)
        def _():
            pltpu.sync_copy(agg_sh, agg_hbm.at[cid])

    return run(inv, ks, kj, d, ew, ex, xw)


def kernel(x, edge_index, edge_weight, edge_color, w1, a1, r1, b1,
           w2, a2, r2, b2):
    x = x.astype(jnp.float32)

    src = edge_index[0].astype(jnp.int32)
    dst = edge_index[1].astype(jnp.int32)
    col = edge_color.astype(jnp.int32)
    pad = E_PAD - E
    kIt = jnp.concatenate([dst * 8 + 2 * col, jnp.zeros((pad,), jnp.int32)])
    kJt = jnp.concatenate([src * 8 + 2 * col + 1,
                           jnp.zeros((pad,), jnp.int32)])
    kJ = jnp.concatenate([col * NS + src, jnp.zeros((pad,), jnp.int32)])
    d_p = jnp.concatenate([dst, jnp.full((pad,), N, jnp.int32)])
    ew_p = jnp.concatenate([edge_weight.astype(jnp.float32),
                            jnp.zeros((pad,), jnp.float32)])
    kS = jnp.concatenate([col * N + dst,
                          jnp.full((pad,), DUMMY_SEG, jnp.int32)])
    kS2 = kS * 2

    xp = jnp.pad(x, ((0, NS - N), (0, 0)))
    a1T = jnp.stack([a1[:, :HID], a1[:, HID:]], axis=2)
    w2p = jnp.pad(w2, ((0, 0), (0, 0), (0, C_PAD - 10)))
    a2T = jnp.pad(jnp.stack([a2[:, :10], a2[:, 10:]], axis=2),
                  ((0, 0), (0, C_PAD - 10), (0, 0)))
    r2p = jnp.pad(r2, ((0, 0), (0, C_PAD - 10)))
    b2p = jnp.pad(b2, ((0, C_PAD - 10)))

    wsc1 = _wprep(w1, a1T, F_IN)
    xw1, sij1, xroot1 = _prep(xp, w1, wsc1, r1, b1.reshape(1, HID), F_IN, HID)
    zrows = jnp.zeros((CHUNK, 2), jnp.float32)
    stats1, ex1 = _sc_pass1(sij1.reshape(-1), kIt, kJt, kS, zrows)
    st1 = stats1.reshape(2, 2 * K_SEG // 128, 128)
    inv1, cntc = _mid1(st1[0], st1[1])
    agg1 = _sc_pass2(inv1.reshape(-1), kS2, kJ, d_p, ew_p, ex1,
                     xw1.reshape(R * NS, HID), HID)

    wsc2 = _wprep(w2p, a2T, HID)
    xw2, sij2, xroot2 = _post1prep2(agg1, xroot1, w2p, wsc2, r2p,
                                    b2p.reshape(1, C_PAD))
    stats2, ex2 = _sc_pass1(sij2.reshape(-1), kIt, kJt, kS, zrows)
    st2 = stats2.reshape(2, 2 * K_SEG // 128, 128)
    inv2 = _mid2(st2[0], st2[1], cntc)
    agg2 = _sc_pass2(inv2.reshape(-1), kS2, kJ, d_p, ew_p, ex2,
                     xw2.reshape(R * NS, C_PAD), C_PAD)
    o = _post2(agg2, xroot2)
    return o[:N, :10]


# speedup vs baseline: 1.1836x; 1.1836x over previous
import dataclasses
import functools

import jax
import jax.numpy as jnp
from jax import lax
from jax.experimental import pallas as pl
from jax.experimental.pallas import tpu as pltpu
from jax.experimental.pallas import tpu_sc as plsc

N = 10000
E = 320000
F_IN = 128
HID = 64
C_PAD = 16
R = 4

NS = 10240
RN = R * NS
DUMMY_KEY = RN
K_SEG = 40064
DUMMY_SEG = R * N
NTILES = 32
CHUNK = 128
NCHUNK = 80
NCH0 = 88
NCH1 = 72
E_PAD = NTILES * NCHUNK * CHUNK
EROWS = E_PAD // CHUNK

_HIGH = lax.Precision.HIGHEST
_mesh = plsc.VectorSubcoreMesh(core_axis_name="c", subcore_axis_name="s",
                               num_cores=2, num_subcores=16)
_sc_params = pltpu.CompilerParams()
if "needs_layout_passes" in pltpu.CompilerParams.__dataclass_fields__:
    _sc_params = dataclasses.replace(_sc_params, needs_layout_passes=False)
if "use_tc_tiling_on_sc" in pltpu.CompilerParams.__dataclass_fields__:
    _sc_params = dataclasses.replace(_sc_params, use_tc_tiling_on_sc=False)


def _wprep_body(w_ref, at_ref, wsc_ref):
    cols = []
    for r in range(R):
        cols.append(jnp.dot(w_ref[r], at_ref[r],
                            preferred_element_type=jnp.float32,
                            precision=_HIGH))
    wsc_ref[...] = jnp.concatenate(cols, axis=1)


def _wprep(w, at, fin):
    return pl.pallas_call(
        _wprep_body,
        out_shape=jax.ShapeDtypeStruct((fin, 2 * R), jnp.float32),
    )(w, at)


def _prep_body(x_ref, w_ref, wsc_ref, root_ref, bias_ref,
               xw_ref, sij_ref, xroot_ref):
    xb = x_ref[...]
    for r in range(R):
        xw_ref[r] = jnp.dot(xb, w_ref[r], preferred_element_type=jnp.float32,
                            precision=_HIGH)
    sij_ref[...] = jnp.dot(xb, wsc_ref[...],
                           preferred_element_type=jnp.float32,
                           precision=_HIGH)
    xroot_ref[...] = (jnp.dot(xb, root_ref[...],
                              preferred_element_type=jnp.float32,
                              precision=_HIGH) + bias_ref[...])


def _prep(x, w, wsc, root, bias2, fin, fo):
    nb = NS // 1024
    return pl.pallas_call(
        _prep_body,
        grid=(nb,),
        in_specs=[
            pl.BlockSpec((1024, fin), lambda i: (i, 0)),
            pl.BlockSpec((R, fin, fo), lambda i: (0, 0, 0)),
            pl.BlockSpec((fin, 2 * R), lambda i: (0, 0)),
            pl.BlockSpec((fin, fo), lambda i: (0, 0)),
            pl.BlockSpec((1, fo), lambda i: (0, 0)),
        ],
        out_specs=[
            pl.BlockSpec((R, 1024, fo), lambda i: (0, i, 0)),
            pl.BlockSpec((1024, 2 * R), lambda i: (i, 0)),
            pl.BlockSpec((1024, fo), lambda i: (i, 0)),
        ],
        out_shape=[
            jax.ShapeDtypeStruct((R, NS, fo), jnp.float32),
            jax.ShapeDtypeStruct((NS, 2 * R), jnp.float32),
            jax.ShapeDtypeStruct((NS, fo), jnp.float32),
        ],
    )(x, w, wsc, root, bias2)


def _mid1_body(da_ref, db_ref, ca_ref, cb_ref, inv_ref, cnt_ref):
    den = da_ref[...] + db_ref[...]
    cntc = jnp.maximum(ca_ref[...] + cb_ref[...], 1.0)
    inv_ref[...] = 1.0 / (den * cntc + 1e-30)
    cnt_ref[...] = cntc


def _mid1(da, db, ca, cb):
    shp = jax.ShapeDtypeStruct((K_SEG // 128, 128), jnp.float32)
    return pl.pallas_call(_mid1_body, out_shape=[shp, shp])(da, db, ca, cb)


def _mid2_body(da_ref, db_ref, cnt_ref, inv_ref):
    den = da_ref[...] + db_ref[...]
    inv_ref[...] = 1.0 / (den * cnt_ref[...] + 1e-30)


def _mid2(da, db, cntc):
    shp = jax.ShapeDtypeStruct((K_SEG // 128, 128), jnp.float32)
    return pl.pallas_call(_mid2_body, out_shape=shp)(da, db, cntc)


def _post1prep2_body(aa_ref, ab_ref, xr_ref, w_ref, wsc_ref, root_ref,
                     bias_ref, xw_ref, sij_ref, xroot_ref):
    h = jnp.maximum(aa_ref[...] + ab_ref[...] + xr_ref[...], 0.0)
    for r in range(R):
        xw_ref[r] = jnp.dot(h, w_ref[r], preferred_element_type=jnp.float32,
                            precision=_HIGH)
    sij_ref[...] = jnp.dot(h, wsc_ref[...],
                           preferred_element_type=jnp.float32,
                           precision=_HIGH)
    xroot_ref[...] = (jnp.dot(h, root_ref[...],
                              preferred_element_type=jnp.float32,
                              precision=_HIGH) + bias_ref[...])


def _post1prep2(agg, xroot, w, wsc, root, bias2):
    nb = NS // 1024
    fo = C_PAD
    spec_h = pl.BlockSpec((1024, HID), lambda i: (i, 0))
    return pl.pallas_call(
        _post1prep2_body,
        grid=(nb,),
        in_specs=[
            spec_h, spec_h, spec_h,
            pl.BlockSpec((R, HID, fo), lambda i: (0, 0, 0)),
            pl.BlockSpec((HID, 2 * R), lambda i: (0, 0)),
            pl.BlockSpec((HID, fo), lambda i: (0, 0)),
            pl.BlockSpec((1, fo), lambda i: (0, 0)),
        ],
        out_specs=[
            pl.BlockSpec((R, 1024, fo), lambda i: (0, i, 0)),
            pl.BlockSpec((1024, 2 * R), lambda i: (i, 0)),
            pl.BlockSpec((1024, fo), lambda i: (i, 0)),
        ],
        out_shape=[
            jax.ShapeDtypeStruct((R, NS, fo), jnp.float32),
            jax.ShapeDtypeStruct((NS, 2 * R), jnp.float32),
            jax.ShapeDtypeStruct((NS, fo), jnp.float32),
        ],
    )(agg[0], agg[1], xroot, w, wsc, root, bias2)


def _post2_body(a_ref, b_ref, xr_ref, o_ref):
    o = a_ref[...] + b_ref[...] + xr_ref[...]
    col = lax.broadcasted_iota(jnp.int32, o.shape, 1)
    om = jnp.where(col < 10, o, -1e30)
    m = jnp.max(om, axis=1, keepdims=True)
    s = jnp.sum(jnp.exp(om - m), axis=1, keepdims=True)
    o_ref[...] = o - m - jnp.log(s)


def _post2(agg, xroot):
    nb = NS // 1024
    spec = pl.BlockSpec((1024, C_PAD), lambda i: (i, 0))
    return pl.pallas_call(
        _post2_body,
        grid=(nb,),
        in_specs=[spec, spec, spec],
        out_specs=spec,
        out_shape=jax.ShapeDtypeStruct((NS, C_PAD), jnp.float32),
    )(agg[0], agg[1], xroot)


_ZERO16 = functools.partial(jnp.zeros, (16,), jnp.float32)
TILE_E = NCHUNK * CHUNK


def _sc_pass1_body(tbl_hbm, ki_hbm, kj_hbm, ks_hbm, z_hbm, stats_out, ex_out,
                   kib, kjb, ksb, ssb, sib, sjb, exb, rows, sems):
    cid = lax.axis_index("c")
    sid = lax.axis_index("s")
    wid = cid * 16 + sid
    semi, semg, semsc = sems

    pltpu.sync_copy(z_hbm, rows.at[0])

    @pl.loop(0, 19)
    def _(i):
        pltpu.sync_copy(rows.at[0], stats_out.at[pl.ds(sid * 2504 + i * CHUNK,
                                                       CHUNK)])
    pltpu.sync_copy(rows.at[0].at[pl.ds(0, 72)],
                    stats_out.at[pl.ds(sid * 2504 + 19 * CHUNK, 72)])
    plsc.subcore_barrier()

    ones16 = jnp.ones((16,), jnp.float32)
    col0 = jnp.zeros((16,), jnp.int32)
    col1 = jnp.ones((16,), jnp.int32)
    base_rows = lax.iota(jnp.int32, 16)

    def pipeline(nch, tbase, rbase):
        def idx_copies(jj, b):
            bs = pl.ds(tbase + jj * CHUNK, CHUNK)
            return [(ki_hbm.at[bs], kib.at[b], semi[b]),
                    (kj_hbm.at[bs], kjb.at[b], semi[b]),
                    (ks_hbm.at[bs], ksb.at[b], semi[b])]

        def g_copies(b):
            return [(tbl_hbm.at[kib.at[b]], sib.at[b], semg[b]),
                    (tbl_hbm.at[kjb.at[b]], sjb.at[b], semg[b])]

        def fire(copies, **kw):
            for src, dst, sm in copies:
                pltpu.async_copy(src, dst, sm, **kw)

        def drain(copies, **kw):
            for src, dst, sm in copies:
                pltpu.make_async_copy(src, dst, sm).wait()

        def s_copy(b):
            return [(rows.at[b], stats_out.at[ssb.at[b]], semsc[b])]

        def compute(j, b):
            for q in range(CHUNK // 16):
                sl = pl.ds(q * 16, 16)
                a = sib[b, sl] + sjb[b, sl]
                a = jnp.where(a >= 0.0, a, 0.2 * a)
                a = jnp.clip(a, -60.0, 60.0)
                e = jnp.exp(a)
                exb[j, sl] = e
                ssb[b, sl] = ksb[b, sl]
                ridx = base_rows + q * 16
                plsc.store_scatter(rows.at[b], [ridx, col0], e)
                plsc.store_scatter(rows.at[b], [ridx, col1], ones16)

        fire(idx_copies(0, 0))
        drain(idx_copies(0, 0))
        fire(g_copies(0))
        fire(idx_copies(1, 1))

        @pl.loop(0, nch // 2)
        def _(pp):
            for b in (0, 1):
                j = pp * 2 + b
                drain(g_copies(b))
                if b == 0:
                    @pl.when(pp >= 1)
                    def _():
                        drain(s_copy(1))
                    drain(idx_copies(j + 1, 1))
                    fire(g_copies(1))
                else:
                    @pl.when(pp < nch // 2 - 1)
                    def _():
                        drain(s_copy(0))
                        drain(idx_copies(j + 1, 0))
                        fire(g_copies(0))
                compute(j, b)
                fire(s_copy(b), add=True)

                @pl.when(pp < nch // 2 - 1)
                def _():
                    fire(idx_copies(j + 2, b))

        drain(s_copy(0))
        drain(s_copy(1))
        pltpu.sync_copy(exb.at[pl.ds(0, nch)],
                        ex_out.at[pl.ds(rbase, nch)])

    @pl.when(cid == 0)
    def _():
        pipeline(NCH0, (sid * NCH0) * CHUNK, sid * NCH0)

    @pl.when(cid == 1)
    def _():
        pipeline(NCH1, (16 * NCH0 + sid * NCH1) * CHUNK, 16 * NCH0 + sid * NCH1)

    plsc.subcore_barrier()


def _sc_pass1(tbl, ki, kj, ks, zrows):
    @functools.partial(
        pl.kernel,
        out_type=[jax.ShapeDtypeStruct((2, K_SEG, 2), jnp.float32),
                  jax.ShapeDtypeStruct((EROWS, CHUNK), jnp.float32)],
        mesh=_mesh,
        scratch_types=[
            pltpu.VMEM((2, CHUNK), jnp.int32),
            pltpu.VMEM((2, CHUNK), jnp.int32),
            pltpu.VMEM((2, CHUNK), jnp.int32),
            pltpu.VMEM((2, CHUNK), jnp.int32),
            pltpu.VMEM((2, CHUNK), jnp.float32),
            pltpu.VMEM((2, CHUNK), jnp.float32),
            pltpu.VMEM((NCH0, CHUNK), jnp.float32),
            pltpu.VMEM((2, CHUNK, 2), jnp.float32),
            pltpu.SemaphoreType.DMA,
            pltpu.SemaphoreType.DMA,
            pltpu.SemaphoreType.DMA,
            pltpu.SemaphoreType.DMA,
            pltpu.SemaphoreType.DMA,
            pltpu.SemaphoreType.DMA,
            pltpu.VMEM_SHARED((K_SEG, 2), jnp.float32),
        ],
        compiler_params=_sc_params,
    )
    def run(tbl_hbm, ki_hbm, kj_hbm, ks_hbm, z_hbm, stats_hbm, ex_hbm,
            kib, kjb, ksb, ssb, sib, sjb, exb, rows,
            si0, si1, sg0, sg1, ss0, ss1, stats_sh):
        cid = lax.axis_index("c")
        sid = lax.axis_index("s")
        sems = ([si0, si1], [sg0, sg1], [ss0, ss1])
        _sc_pass1_body(tbl_hbm, ki_hbm, kj_hbm, ks_hbm, z_hbm, stats_sh,
                       ex_hbm, kib, kjb, ksb, ssb, sib, sjb, exb, rows, sems)

        @pl.when(sid == 0)
        def _():
            pltpu.sync_copy(stats_sh, stats_hbm.at[cid])

    return run(tbl, ki, kj, ks, zrows)


def _sc_pass2_body(inv_hbm, ks_hbm, kj_hbm, d_hbm, ew_hbm, ex_hbm, xw_hbm,
                   agg_out, ksb, kjb, db, sdb, ewc, exc, invc, coefb, rows,
                   sems, fo):
    cid = lax.axis_index("c")
    sid = lax.axis_index("s")
    wid = cid * 16 + sid
    nt = fo // 16
    semi, semg, semsc = sems

    @pl.loop(0, CHUNK)
    def _(i):
        for t in range(nt):
            rows[0, i, pl.ds(t * 16, 16)] = _ZERO16()

    @pl.loop(0, NS // 16 // CHUNK)
    def _(i):
        pltpu.sync_copy(rows.at[0],
                        agg_out.at[pl.ds(sid * (NS // 16) + i * CHUNK, CHUNK)])
    plsc.subcore_barrier()

    def pipeline(nch, tbase):
        def idx_copies(jj, b):
            bs = pl.ds(tbase + jj * CHUNK, CHUNK)
            return [(ks_hbm.at[bs], ksb.at[b], semi[b]),
                    (kj_hbm.at[bs], kjb.at[b], semi[b]),
                    (d_hbm.at[bs], db.at[b], semi[b]),
                    (ew_hbm.at[bs], ewc.at[b], semi[b]),
                    (ex_hbm.at[bs], exc.at[b], semi[b])]

        def g_copies(b):
            return [(inv_hbm.at[ksb.at[b]], invc.at[b], semg[b]),
                    (xw_hbm.at[kjb.at[b]], rows.at[b], semg[b])]

        def fire(copies, **kw):
            for src, dst, sm in copies:
                pltpu.async_copy(src, dst, sm, **kw)

        def drain(copies, **kw):
            for src, dst, sm in copies:
                pltpu.make_async_copy(src, dst, sm).wait()

        def s_copy(b):
            return [(rows.at[b], agg_out.at[sdb.at[b]], semsc[b])]

        def compute(b):
            for q in range(CHUNK // 16):
                sl = pl.ds(q * 16, 16)
                coefb[sl] = ewc[b, sl] * exc[b, sl] * invc[b, sl]
                sdb[b, sl] = db[b, sl]
            for jj in range(CHUNK):
                cv = plsc.load_gather(coefb, [jnp.full((16,), jj, jnp.int32)])
                for t in range(nt):
                    slt = pl.ds(t * 16, 16)
                    rows[b, jj, slt] = rows[b, jj, slt] * cv

        fire(idx_copies(0, 0))
        drain(idx_copies(0, 0))
        fire(g_copies(0))
        fire(idx_copies(1, 1))

        @pl.loop(0, nch // 2)
        def _(pp):
            for b in (0, 1):
                j = pp * 2 + b
                drain(g_copies(b))
                if b == 0:
                    @pl.when(pp >= 1)
                    def _():
                        drain(s_copy(1))
                    drain(idx_copies(j + 1, 1))
                    fire(g_copies(1))
                else:
                    @pl.when(pp < nch // 2 - 1)
                    def _():
                        drain(s_copy(0))
                        drain(idx_copies(j + 1, 0))
                        fire(g_copies(0))
                compute(b)
                fire(s_copy(b), add=True)

                @pl.when(pp < nch // 2 - 1)
                def _():
                    fire(idx_copies(j + 2, b))

        drain(s_copy(0))
        drain(s_copy(1))

    @pl.when(cid == 0)
    def _():
        pipeline(NCH0, (sid * NCH0) * CHUNK)

    @pl.when(cid == 1)
    def _():
        pipeline(NCH1, (16 * NCH0 + sid * NCH1) * CHUNK)

    plsc.subcore_barrier()


def _sc_pass2(inv, ks, kj, d, ew, ex, xw, fo):
    @functools.partial(
        pl.kernel,
        out_type=jax.ShapeDtypeStruct((2, NS, fo), jnp.float32),
        mesh=_mesh,
        scratch_types=[
            pltpu.VMEM((2, CHUNK), jnp.int32),
            pltpu.VMEM((2, CHUNK), jnp.int32),
            pltpu.VMEM((2, CHUNK), jnp.int32),
            pltpu.VMEM((2, CHUNK), jnp.int32),
            pltpu.VMEM((2, CHUNK), jnp.float32),
            pltpu.VMEM((2, CHUNK), jnp.float32),
            pltpu.VMEM((2, CHUNK), jnp.float32),
            pltpu.VMEM((CHUNK,), jnp.float32),
            pltpu.VMEM((2, CHUNK, fo), jnp.float32),
            pltpu.SemaphoreType.DMA,
            pltpu.SemaphoreType.DMA,
            pltpu.SemaphoreType.DMA,
            pltpu.SemaphoreType.DMA,
            pltpu.SemaphoreType.DMA,
            pltpu.SemaphoreType.DMA,
            pltpu.VMEM_SHARED((NS, fo), jnp.float32),
        ],
        compiler_params=_sc_params,
    )
    def run(inv_hbm, ks_hbm, kj_hbm, d_hbm, ew_hbm, ex_hbm, xw_hbm, agg_hbm,
            ksb, kjb, db, sdb, ewc, exc, invc, coefb, rows,
            si0, si1, sg0, sg1, ss0, ss1, agg_sh):
        cid = lax.axis_index("c")
        sid = lax.axis_index("s")
        sems = ([si0, si1], [sg0, sg1], [ss0, ss1])
        _sc_pass2_body(inv_hbm, ks_hbm, kj_hbm, d_hbm, ew_hbm, ex_hbm,
                       xw_hbm, agg_sh, ksb, kjb, db, sdb, ewc, exc, invc,
                       coefb, rows, sems, fo)

        @pl.when(sid == 0)
        def _():
            pltpu.sync_copy(agg_sh, agg_hbm.at[cid])

    return run(inv, ks, kj, d, ew, ex, xw)


def kernel(x, edge_index, edge_weight, edge_color, w1, a1, r1, b1,
           w2, a2, r2, b2):
    x = x.astype(jnp.float32)

    src = edge_index[0].astype(jnp.int32)
    dst = edge_index[1].astype(jnp.int32)
    col = edge_color.astype(jnp.int32)
    pad = E_PAD - E
    kIt = jnp.concatenate([dst * 8 + 2 * col, jnp.zeros((pad,), jnp.int32)])
    kJt = jnp.concatenate([src * 8 + 2 * col + 1,
                           jnp.zeros((pad,), jnp.int32)])
    kJ = jnp.concatenate([col * NS + src, jnp.zeros((pad,), jnp.int32)])
    d_p = jnp.concatenate([dst, jnp.full((pad,), N, jnp.int32)])
    ew_p = jnp.concatenate([edge_weight.astype(jnp.float32),
                            jnp.zeros((pad,), jnp.float32)])
    kS = jnp.concatenate([col * N + dst,
                          jnp.full((pad,), DUMMY_SEG, jnp.int32)])

    xp = jnp.pad(x, ((0, NS - N), (0, 0)))
    a1T = jnp.stack([a1[:, :HID], a1[:, HID:]], axis=2)
    w2p = jnp.pad(w2, ((0, 0), (0, 0), (0, C_PAD - 10)))
    a2T = jnp.pad(jnp.stack([a2[:, :10], a2[:, 10:]], axis=2),
                  ((0, 0), (0, C_PAD - 10), (0, 0)))
    r2p = jnp.pad(r2, ((0, 0), (0, C_PAD - 10)))
    b2p = jnp.pad(b2, ((0, C_PAD - 10)))

    wsc1 = _wprep(w1, a1T, F_IN)
    xw1, sij1, xroot1 = _prep(xp, w1, wsc1, r1, b1.reshape(1, HID), F_IN, HID)
    zrows = jnp.zeros((CHUNK, 2), jnp.float32)
    stats1, ex1 = _sc_pass1(sij1.reshape(-1), kIt, kJt, kS, zrows)
    sd1 = stats1[:, :, 0].reshape(2, K_SEG // 128, 128)
    sc1 = stats1[:, :, 1].reshape(2, K_SEG // 128, 128)
    inv1, cntc = _mid1(sd1[0], sd1[1], sc1[0], sc1[1])
    agg1 = _sc_pass2(inv1.reshape(-1), kS, kJ, d_p, ew_p, ex1.reshape(-1),
                     xw1.reshape(R * NS, HID), HID)

    wsc2 = _wprep(w2p, a2T, HID)
    xw2, sij2, xroot2 = _post1prep2(agg1, xroot1, w2p, wsc2, r2p,
                                    b2p.reshape(1, C_PAD))
    stats2, ex2 = _sc_pass1(sij2.reshape(-1), kIt, kJt, kS, zrows)
    sd2 = stats2[:, :, 0].reshape(2, K_SEG // 128, 128)
    inv2 = _mid2(sd2[0], sd2[1], cntc)
    agg2 = _sc_pass2(inv2.reshape(-1), kS, kJ, d_p, ew_p, ex2.reshape(-1),
                     xw2.reshape(R * NS, C_PAD), C_PAD)
    o = _post2(agg2, xroot2)
    return o[:N, :10]
